# Initial kernel scaffold; baseline (speedup 1.0000x reference)
import functools

import jax
import jax.numpy as jnp
from jax import lax
from jax.experimental import pallas as pl
from jax.experimental.pallas import tpu as pltpu
from jax.experimental.pallas import tpu_sc as plsc

N = 10000
E = 320000
D = 128
LANES = 16
NC = 2
NS = 16
NW = NC * NS
EPW = E // NW
C = 80
NCHUNK = EPW // C
STRIPE = N // NS
ZR = 125


def _sc_edge_body(a_hbm, b_hbm, src_hbm, rel_hbm, dst_hbm,
                  aggp_hbm, degp_hbm,
                  agg_sh, deg_sh, srcv, relv, dstv, rows_a, rows_b,
                  onesv, zrow, zdeg, sem_a, sem_b):
    c = lax.axis_index("c")
    s = lax.axis_index("s")
    w = c * NS + s

    def fill_zrow(i, carry):
        for j in range(D // LANES):
            zrow[i, pl.ds(j * LANES, LANES)] = jnp.zeros((LANES,), jnp.float32)
        zdeg[i, :] = jnp.zeros((LANES,), jnp.float32)
        return carry
    lax.fori_loop(0, ZR, fill_zrow, 0)

    def fill_ones(i, carry):
        onesv[i, :] = jnp.ones((LANES,), jnp.float32)
        return carry
    lax.fori_loop(0, C, fill_ones, 0)

    base = s * STRIPE
    for k in range(STRIPE // ZR):
        pltpu.sync_copy(zrow, agg_sh.at[pl.ds(base + k * ZR, ZR)])
        pltpu.sync_copy(zdeg, deg_sh.at[pl.ds(base + k * ZR, ZR)])
    plsc.subcore_barrier()

    def chunk(g, carry):
        base_e = w * EPW + g * C
        pltpu.sync_copy(src_hbm.at[pl.ds(base_e, C)], srcv)
        pltpu.sync_copy(rel_hbm.at[pl.ds(base_e, C)], relv)
        pltpu.sync_copy(dst_hbm.at[pl.ds(base_e, C)], dstv)
        cp_a = pltpu.async_copy(a_hbm.at[srcv], rows_a, sem_a)
        cp_b = pltpu.async_copy(b_hbm.at[relv], rows_b, sem_b)
        cp_a.wait()
        cp_b.wait()

        def edge(e, inner):
            for j in range(D // LANES):
                sl = pl.ds(j * LANES, LANES)
                v = rows_a[e, sl] + rows_b[e, sl]
                rows_a[e, sl] = jnp.maximum(v, 0.0)
            return inner
        lax.fori_loop(0, C, edge, 0)

        pltpu.sync_copy(rows_a, agg_sh.at[dstv], add=True)
        pltpu.sync_copy(onesv, deg_sh.at[dstv], add=True)
        return carry
    lax.fori_loop(0, NCHUNK, chunk, 0)

    plsc.subcore_barrier()

    pltpu.sync_copy(agg_sh.at[pl.ds(base, STRIPE)],
                    aggp_hbm.at[c, pl.ds(base, STRIPE)])
    pltpu.sync_copy(deg_sh.at[pl.ds(base, STRIPE)],
                    degp_hbm.at[c, pl.ds(base, STRIPE)])


_sc_edge = functools.partial(
    pl.kernel,
    out_type=[jax.ShapeDtypeStruct((NC, N, D), jnp.float32),
              jax.ShapeDtypeStruct((NC, N, LANES), jnp.float32)],
    mesh=plsc.VectorSubcoreMesh(core_axis_name="c", subcore_axis_name="s"),
    scratch_types=[
        pltpu.MemoryRef((N, D), jnp.float32, pltpu.MemorySpace.VMEM_SHARED),
        pltpu.MemoryRef((N, LANES), jnp.float32, pltpu.MemorySpace.VMEM_SHARED),
        pltpu.VMEM((C,), jnp.int32),
        pltpu.VMEM((C,), jnp.int32),
        pltpu.VMEM((C,), jnp.int32),
        pltpu.VMEM((C, D), jnp.float32),
        pltpu.VMEM((C, D), jnp.float32),
        pltpu.VMEM((C, LANES), jnp.float32),
        pltpu.VMEM((ZR, D), jnp.float32),
        pltpu.VMEM((ZR, LANES), jnp.float32),
        pltpu.SemaphoreType.DMA,
        pltpu.SemaphoreType.DMA,
    ],
)(_sc_edge_body)


def _mm_body(x_ref, w_ref, o_ref):
    o_ref[...] = jnp.dot(x_ref[...], w_ref[...],
                         preferred_element_type=jnp.float32)


def _matmul(x, w, block_rows):
    m, k = x.shape
    _, n = w.shape
    return pl.pallas_call(
        _mm_body,
        grid=(m // block_rows,),
        in_specs=[pl.BlockSpec((block_rows, k), lambda i: (i, 0)),
                  pl.BlockSpec((k, n), lambda i: (0, 0))],
        out_specs=pl.BlockSpec((block_rows, n), lambda i: (i, 0)),
        out_shape=jax.ShapeDtypeStruct((m, n), jnp.float32),
    )(x, w)


def _out_body(ent_ref, aggp_ref, degp_ref, w1_ref, w2_ref, o_ref):
    agg = aggp_ref[0] + aggp_ref[1]
    deg = degp_ref[0, :, 0:1] + degp_ref[1, :, 0:1]
    aggn = agg / jnp.maximum(deg, 1.0)
    h = jnp.dot(ent_ref[...], w1_ref[...], preferred_element_type=jnp.float32)
    h = h + jnp.dot(aggn, w2_ref[...], preferred_element_type=jnp.float32)
    o_ref[...] = jnp.maximum(h, 0.0)


def _node_update(ent, aggp, degp, w1, w2, block_rows):
    m = ent.shape[0]
    return pl.pallas_call(
        _out_body,
        grid=(m // block_rows,),
        in_specs=[
            pl.BlockSpec((block_rows, D), lambda i: (i, 0)),
            pl.BlockSpec((NC, block_rows, D), lambda i: (0, i, 0)),
            pl.BlockSpec((NC, block_rows, LANES), lambda i: (0, i, 0)),
            pl.BlockSpec((D, D), lambda i: (0, 0)),
            pl.BlockSpec((D, D), lambda i: (0, 0)),
        ],
        out_specs=pl.BlockSpec((block_rows, D), lambda i: (i, 0)),
        out_shape=jax.ShapeDtypeStruct((m, D), jnp.float32),
    )(ent, aggp, degp, w1, w2)


def kernel(ent_embeds, rel_embeds, W_msg, W_out, edge_index, edge_rel):
    src = edge_index[0]
    dst = edge_index[1]
    a_tab = _matmul(ent_embeds, W_msg[:D], 1000)
    b_tab = _matmul(rel_embeds, W_msg[D:], 256)
    aggp, degp = _sc_edge(a_tab, b_tab, src, edge_rel, dst)
    return _node_update(ent_embeds, aggp, degp, W_out[:D], W_out[D:], 1000)


# speedup vs baseline: 5.4550x; 5.4550x over previous
import functools

import jax
import jax.numpy as jnp
from jax import lax
from jax.experimental import pallas as pl
from jax.experimental.pallas import tpu as pltpu
from jax.experimental.pallas import tpu_sc as plsc

N = 10000
E = 320000
D = 128
LANES = 16
NC = 2
NS = 16
NW = NC * NS
EPW = E // NW
C = 80
NCHUNK = EPW // C
STRIPE = 640
TAIL = N - 15 * STRIPE
ZR = 80


def _sc_edge_body(a_hbm, b_hbm, src_hbm, rel_hbm, dst_hbm,
                  aggp_hbm, degp_hbm,
                  agg_sh, srcv, relv, dstv, rows_a, rows_b,
                  degv, zrow, sem_a, sem_b):
    c = lax.axis_index("c")
    s = lax.axis_index("s")
    w = c * NS + s

    def fill_zrow(i, carry):
        for j in range(D // LANES):
            zrow[i, pl.ds(j * LANES, LANES)] = jnp.zeros((LANES,), jnp.float32)
        return carry
    lax.fori_loop(0, ZR, fill_zrow, 0)

    def zero_deg(i, carry):
        degv[pl.ds(i * LANES, LANES)] = jnp.zeros((LANES,), jnp.float32)
        return carry
    lax.fori_loop(0, N // LANES, zero_deg, 0)

    base = s * STRIPE
    nz = lax.select(s < 15, STRIPE // ZR, TAIL // ZR)

    def zero_stripe(k, carry):
        pltpu.sync_copy(zrow, agg_sh.at[pl.ds(base + k * ZR, ZR)])
        return carry
    lax.fori_loop(0, nz, zero_stripe, 0)
    plsc.subcore_barrier()

    def chunk(g, carry):
        base_e = w * EPW + g * C
        pltpu.sync_copy(src_hbm.at[pl.ds(base_e, C)], srcv)
        pltpu.sync_copy(rel_hbm.at[pl.ds(base_e, C)], relv)
        pltpu.sync_copy(dst_hbm.at[pl.ds(base_e, C)], dstv)
        cp_a = pltpu.async_copy(a_hbm.at[srcv], rows_a, sem_a)
        cp_b = pltpu.async_copy(b_hbm.at[relv], rows_b, sem_b)

        lane = lax.iota(jnp.int32, LANES)
        one = jnp.ones((LANES,), jnp.float32)
        for k in range(C // LANES):
            d16 = dstv[pl.ds(k * LANES, LANES)]
            for l in range(LANES):
                plsc.addupdate_scatter(degv, [d16], one, mask=lane == l)

        cp_a.wait()
        cp_b.wait()

        def edge(e, inner):
            for j in range(D // LANES):
                sl = pl.ds(j * LANES, LANES)
                v = rows_a[e, sl] + rows_b[e, sl]
                rows_a[e, sl] = jnp.maximum(v, 0.0)
            return inner
        lax.fori_loop(0, C, edge, 0)

        pltpu.sync_copy(rows_a, agg_sh.at[dstv], add=True)
        return carry
    lax.fori_loop(0, NCHUNK, chunk, 0)

    plsc.subcore_barrier()

    @pl.when(s < 15)
    def _():
        pltpu.sync_copy(agg_sh.at[pl.ds(base, STRIPE)],
                        aggp_hbm.at[c, pl.ds(base, STRIPE)])

    @pl.when(s == 15)
    def _():
        pltpu.sync_copy(agg_sh.at[pl.ds(15 * STRIPE, TAIL)],
                        aggp_hbm.at[c, pl.ds(15 * STRIPE, TAIL)])

    pltpu.sync_copy(degv, degp_hbm.at[pl.ds(w * N, N)])


_sc_edge = functools.partial(
    pl.kernel,
    out_type=[jax.ShapeDtypeStruct((NC, N, D), jnp.float32),
              jax.ShapeDtypeStruct((NW * N,), jnp.float32)],
    mesh=plsc.VectorSubcoreMesh(core_axis_name="c", subcore_axis_name="s"),
    compiler_params=pltpu.CompilerParams(needs_layout_passes=False),
    scratch_types=[
        pltpu.VMEM_SHARED((N, D), jnp.float32),
        pltpu.VMEM((C,), jnp.int32),
        pltpu.VMEM((C,), jnp.int32),
        pltpu.VMEM((C,), jnp.int32),
        pltpu.VMEM((C, D), jnp.float32),
        pltpu.VMEM((C, D), jnp.float32),
        pltpu.VMEM((N,), jnp.float32),
        pltpu.VMEM((ZR, D), jnp.float32),
        pltpu.SemaphoreType.DMA,
        pltpu.SemaphoreType.DMA,
    ],
)(_sc_edge_body)


def _mm_body(x_ref, w_ref, o_ref):
    o_ref[...] = jnp.dot(x_ref[...], w_ref[...],
                         preferred_element_type=jnp.float32)


def _matmul(x, w, block_rows):
    m, k = x.shape
    _, n = w.shape
    return pl.pallas_call(
        _mm_body,
        grid=(m // block_rows,),
        in_specs=[pl.BlockSpec((block_rows, k), lambda i: (i, 0)),
                  pl.BlockSpec((k, n), lambda i: (0, 0))],
        out_specs=pl.BlockSpec((block_rows, n), lambda i: (i, 0)),
        out_shape=jax.ShapeDtypeStruct((m, n), jnp.float32),
    )(x, w)


def _out_body(ent_ref, aggp_ref, degp_ref, w1_ref, w2_ref, o_ref):
    agg = aggp_ref[0] + aggp_ref[1]
    deg = jnp.sum(degp_ref[...], axis=1, keepdims=True)
    aggn = agg / jnp.maximum(deg, 1.0)
    h = jnp.dot(ent_ref[...], w1_ref[...], preferred_element_type=jnp.float32)
    h = h + jnp.dot(aggn, w2_ref[...], preferred_element_type=jnp.float32)
    o_ref[...] = jnp.maximum(h, 0.0)


def _node_update(ent, aggp, degp, w1, w2, block_rows):
    m = ent.shape[0]
    return pl.pallas_call(
        _out_body,
        grid=(m // block_rows,),
        in_specs=[
            pl.BlockSpec((block_rows, D), lambda i: (i, 0)),
            pl.BlockSpec((NC, block_rows, D), lambda i: (0, i, 0)),
            pl.BlockSpec((block_rows, NW), lambda i: (i, 0)),
            pl.BlockSpec((D, D), lambda i: (0, 0)),
            pl.BlockSpec((D, D), lambda i: (0, 0)),
        ],
        out_specs=pl.BlockSpec((block_rows, D), lambda i: (i, 0)),
        out_shape=jax.ShapeDtypeStruct((m, D), jnp.float32),
    )(ent, aggp, degp, w1, w2)


def kernel(ent_embeds, rel_embeds, W_msg, W_out, edge_index, edge_rel):
    src = edge_index[0]
    dst = edge_index[1]
    a_tab = _matmul(ent_embeds, W_msg[:D], 1000)
    b_tab = _matmul(rel_embeds, W_msg[D:], 256)
    aggp, degflat = _sc_edge(a_tab, b_tab, src, edge_rel, dst)
    degp = degflat.reshape(NW, N).T
    return _node_update(ent_embeds, aggp, degp, W_out[:D], W_out[D:], 1000)
